# baseline (speedup 1.0000x reference)
import functools

import jax
import jax.numpy as jnp
from jax import lax
from jax.experimental import pallas as pl
from jax.experimental.pallas import tpu as pltpu
from jax.experimental.pallas import tpu_sc as plsc

_NC = 2
_NS = 16
_NW = _NC * _NS


def _sc_gather(n_tot: int, d: int, ch: int):
    n_per_w = n_tot // _NW
    nch = n_per_w // ch
    mesh = plsc.VectorSubcoreMesh(core_axis_name="c", subcore_axis_name="s")

    @functools.partial(
        pl.kernel,
        mesh=mesh,
        out_type=jax.ShapeDtypeStruct((n_tot, d), jnp.float32),
        scratch_types=[
            pltpu.VMEM((n_per_w,), jnp.int32),
            pltpu.VMEM((ch, d), jnp.float32),
            pltpu.VMEM((ch, d), jnp.float32),
            pltpu.SemaphoreType.DMA,
            pltpu.SemaphoreType.DMA,
            pltpu.SemaphoreType.DMA,
            pltpu.SemaphoreType.DMA,
        ],
    )
    def k(idx_hbm, table_hbm, out_hbm, idx_v, rows0, rows1, g0, g1, w0, w1):
        wid = lax.axis_index("s") * _NC + lax.axis_index("c")
        base = wid * n_per_w
        pltpu.sync_copy(idx_hbm.at[pl.ds(base, n_per_w)], idx_v)
        bufs = (rows0, rows1)
        gsem = (g0, g1)
        wsem = (w0, w1)

        def gather_start(c):
            idx_c = idx_v.at[pl.ds(c * ch, ch)]
            pltpu.async_copy(table_hbm.at[idx_c], bufs[c % 2], gsem[c % 2])

        def write_start(c):
            pltpu.async_copy(
                bufs[c % 2], out_hbm.at[pl.ds(base + c * ch, ch)], wsem[c % 2]
            )

        gather_start(0)
        for c in range(nch):
            pltpu.make_async_copy(
                table_hbm.at[idx_v.at[pl.ds(c * ch, ch)]], bufs[c % 2], gsem[c % 2]
            ).wait()
            write_start(c)
            if c + 1 < nch:
                if c >= 1:
                    pltpu.make_async_copy(
                        bufs[(c + 1) % 2],
                        out_hbm.at[pl.ds(base + (c - 1) * ch, ch)],
                        wsem[(c + 1) % 2],
                    ).wait()
                gather_start(c + 1)
        pltpu.make_async_copy(
            bufs[(nch - 1) % 2],
            out_hbm.at[pl.ds(base + (nch - 1) * ch, ch)],
            wsem[(nch - 1) % 2],
        ).wait()
        if nch >= 2:
            pltpu.make_async_copy(
                bufs[(nch - 2) % 2],
                out_hbm.at[pl.ds(base + (nch - 2) * ch, ch)],
                wsem[(nch - 2) % 2],
            ).wait()

    return k


def _tc_head(n_tot: int, d: int, v: int, r: int):
    nblk = n_tot // r

    def body(tok_ref, pos_ref, w_ref, b_ref, out_ref):
        h = tok_ref[...] + pos_ref[...]
        out_ref[...] = (
            jnp.dot(h, w_ref[...], preferred_element_type=jnp.float32)
            + b_ref[...]
        )

    return pl.pallas_call(
        body,
        grid=(nblk,),
        in_specs=[
            pl.BlockSpec((r, d), lambda i: (i, 0)),
            pl.BlockSpec((r, d), lambda i: (0, 0)),
            pl.BlockSpec((d, v), lambda i: (0, 0)),
            pl.BlockSpec((1, v), lambda i: (0, 0)),
        ],
        out_specs=pl.BlockSpec((r, v), lambda i: (i, 0)),
        out_shape=jax.ShapeDtypeStruct((n_tot, v), jnp.float32),
    )


def kernel(x, tok_table, pos_table, W, b):
    bx, tx = x.shape
    vocab, d = tok_table.shape
    n_tot = bx * tx
    dp = 128

    idx = x.reshape(n_tot).astype(jnp.int32)
    tok_pad = jnp.pad(tok_table, ((0, 0), (0, dp - d)))
    tok_emb = _sc_gather(n_tot, dp, ch=400)(idx, tok_pad)

    r = 64 * tx
    w_pad = jnp.pad(W, ((0, dp - d), (0, 0)))
    pos_tiled = jnp.tile(jnp.pad(pos_table, ((0, 0), (0, dp - d))), (r // tx, 1))
    logits = _tc_head(n_tot, dp, vocab, r)(
        tok_emb, pos_tiled, w_pad, b.reshape(1, vocab)
    )
    return logits.reshape(bx, tx, vocab)


# speedup vs baseline: 1.2348x; 1.2348x over previous
import functools

import jax
import jax.numpy as jnp
from jax import lax
from jax.experimental import pallas as pl
from jax.experimental.pallas import tpu as pltpu
from jax.experimental.pallas import tpu_sc as plsc

_NC = 2
_NS = 16
_NW = _NC * _NS


def _sc_gather(n_tot: int, d: int, ch: int):
    n_per_w = n_tot // _NW
    nch = n_per_w // ch
    mesh = plsc.VectorSubcoreMesh(core_axis_name="c", subcore_axis_name="s")

    @functools.partial(
        pl.kernel,
        mesh=mesh,
        out_type=jax.ShapeDtypeStruct((n_tot, d), jnp.float32),
        scratch_types=[
            pltpu.VMEM((n_per_w,), jnp.int32),
            pltpu.VMEM((ch, d), jnp.float32),
            pltpu.VMEM((ch, d), jnp.float32),
            pltpu.SemaphoreType.DMA,
            pltpu.SemaphoreType.DMA,
            pltpu.SemaphoreType.DMA,
            pltpu.SemaphoreType.DMA,
        ],
    )
    def k(idx_hbm, table_hbm, out_hbm, idx_v, rows0, rows1, g0, g1, w0, w1):
        wid = lax.axis_index("s") * _NC + lax.axis_index("c")
        base = wid * n_per_w
        pltpu.sync_copy(idx_hbm.at[pl.ds(base, n_per_w)], idx_v)
        bufs = (rows0, rows1)
        gsem = (g0, g1)
        wsem = (w0, w1)

        def gather_start(c):
            idx_c = idx_v.at[pl.ds(c * ch, ch)]
            pltpu.async_copy(table_hbm.at[idx_c], bufs[c % 2], gsem[c % 2])

        def write_start(c):
            pltpu.async_copy(
                bufs[c % 2], out_hbm.at[pl.ds(base + c * ch, ch)], wsem[c % 2]
            )

        gather_start(0)
        for c in range(nch):
            pltpu.make_async_copy(
                table_hbm.at[idx_v.at[pl.ds(c * ch, ch)]], bufs[c % 2], gsem[c % 2]
            ).wait()
            write_start(c)
            if c + 1 < nch:
                if c >= 1:
                    pltpu.make_async_copy(
                        bufs[(c + 1) % 2],
                        out_hbm.at[pl.ds(base + (c - 1) * ch, ch)],
                        wsem[(c + 1) % 2],
                    ).wait()
                gather_start(c + 1)
        pltpu.make_async_copy(
            bufs[(nch - 1) % 2],
            out_hbm.at[pl.ds(base + (nch - 1) * ch, ch)],
            wsem[(nch - 1) % 2],
        ).wait()
        if nch >= 2:
            pltpu.make_async_copy(
                bufs[(nch - 2) % 2],
                out_hbm.at[pl.ds(base + (nch - 2) * ch, ch)],
                wsem[(nch - 2) % 2],
            ).wait()

    return k


def _tc_head(bx: int, tx: int, d: int, v: int, g: int):
    nblk = bx // g

    def body(tok_ref, pos_ref, w_ref, b_ref, out_ref):
        w = w_ref[...]
        bias = b_ref[...]
        pos = pos_ref[...]
        for j in range(g):
            h = tok_ref[pl.ds(j * tx, tx), :] + pos
            out_ref[j] = (
                jnp.dot(h, w, preferred_element_type=jnp.float32) + bias
            )

    return pl.pallas_call(
        body,
        grid=(nblk,),
        in_specs=[
            pl.BlockSpec((g * tx, d), lambda i: (i, 0)),
            pl.BlockSpec((tx, d), lambda i: (0, 0)),
            pl.BlockSpec((d, v), lambda i: (0, 0)),
            pl.BlockSpec((1, v), lambda i: (0, 0)),
        ],
        out_specs=pl.BlockSpec((g, tx, v), lambda i: (i, 0, 0)),
        out_shape=jax.ShapeDtypeStruct((bx, tx, v), jnp.float32),
    )


def kernel(x, tok_table, pos_table, W, b):
    bx, tx = x.shape
    vocab, d = tok_table.shape
    n_tot = bx * tx
    dp = 128

    idx = x.reshape(n_tot).astype(jnp.int32)
    tok_pad = jnp.pad(tok_table, ((0, 0), (0, dp - d)))
    tok_emb = _sc_gather(n_tot, dp, ch=400)(idx, tok_pad)

    w_pad = jnp.pad(W, ((0, dp - d), (0, 0)))
    pos_pad = jnp.pad(pos_table, ((0, 0), (0, dp - d)))
    return _tc_head(bx, tx, dp, vocab, g=16)(
        tok_emb, pos_pad, w_pad, b.reshape(1, vocab)
    )


# speedup vs baseline: 1.3105x; 1.0613x over previous
import functools

import jax
import jax.numpy as jnp
from jax import lax
from jax.experimental import pallas as pl
from jax.experimental.pallas import tpu as pltpu
from jax.experimental.pallas import tpu_sc as plsc

_NC = 2
_NS = 16
_NW = _NC * _NS


def _sc_gather(n_tot: int, d: int, ch: int):
    n_per_w = n_tot // _NW
    nch = n_per_w // ch
    mesh = plsc.VectorSubcoreMesh(core_axis_name="c", subcore_axis_name="s")

    @functools.partial(
        pl.kernel,
        mesh=mesh,
        out_type=jax.ShapeDtypeStruct((n_tot, d), jnp.float32),
        scratch_types=[
            pltpu.VMEM((n_per_w,), jnp.int32),
            pltpu.VMEM((ch, d), jnp.float32),
            pltpu.VMEM((ch, d), jnp.float32),
            pltpu.SemaphoreType.DMA,
            pltpu.SemaphoreType.DMA,
            pltpu.SemaphoreType.DMA,
            pltpu.SemaphoreType.DMA,
        ],
    )
    def k(idx_hbm, table_hbm, out_hbm, idx_v, rows0, rows1, g0, g1, w0, w1):
        wid = lax.axis_index("s") * _NC + lax.axis_index("c")
        base = wid * n_per_w
        pltpu.sync_copy(idx_hbm.at[pl.ds(base, n_per_w)], idx_v)
        bufs = (rows0, rows1)
        gsem = (g0, g1)
        wsem = (w0, w1)

        def gather_start(c):
            idx_c = idx_v.at[pl.ds(c * ch, ch)]
            pltpu.async_copy(table_hbm.at[idx_c], bufs[c % 2], gsem[c % 2])

        def write_start(c):
            pltpu.async_copy(
                bufs[c % 2], out_hbm.at[pl.ds(base + c * ch, ch)], wsem[c % 2]
            )

        gather_start(0)
        for c in range(nch):
            pltpu.make_async_copy(
                table_hbm.at[idx_v.at[pl.ds(c * ch, ch)]], bufs[c % 2], gsem[c % 2]
            ).wait()
            write_start(c)
            if c + 1 < nch:
                if c >= 1:
                    pltpu.make_async_copy(
                        bufs[(c + 1) % 2],
                        out_hbm.at[pl.ds(base + (c - 1) * ch, ch)],
                        wsem[(c + 1) % 2],
                    ).wait()
                gather_start(c + 1)
        pltpu.make_async_copy(
            bufs[(nch - 1) % 2],
            out_hbm.at[pl.ds(base + (nch - 1) * ch, ch)],
            wsem[(nch - 1) % 2],
        ).wait()
        if nch >= 2:
            pltpu.make_async_copy(
                bufs[(nch - 2) % 2],
                out_hbm.at[pl.ds(base + (nch - 2) * ch, ch)],
                wsem[(nch - 2) % 2],
            ).wait()

    return k


def _tc_head(bx: int, tx: int, d: int, v: int, g: int):
    nblk = bx // g

    def body(tok_ref, pos_ref, w_ref, b_ref, out_ref):
        w = w_ref[...]
        bias = b_ref[...]
        pos = pos_ref[...]
        for j in range(g):
            h = tok_ref[pl.ds(j * tx, tx), :] + pos
            out_ref[j] = (
                jnp.dot(h, w, preferred_element_type=jnp.float32) + bias
            )

    return pl.pallas_call(
        body,
        grid=(nblk,),
        in_specs=[
            pl.BlockSpec((g * tx, d), lambda i: (i, 0)),
            pl.BlockSpec((tx, d), lambda i: (0, 0)),
            pl.BlockSpec((d, v), lambda i: (0, 0)),
            pl.BlockSpec((1, v), lambda i: (0, 0)),
        ],
        out_specs=pl.BlockSpec((g, tx, v), lambda i: (i, 0, 0)),
        out_shape=jax.ShapeDtypeStruct((bx, tx, v), jnp.float32),
    )


def kernel(x, tok_table, pos_table, W, b):
    bx, tx = x.shape
    vocab, d = tok_table.shape
    n_tot = bx * tx
    dp = 128

    idx = x.reshape(n_tot).astype(jnp.int32)
    tok_pad = jnp.pad(tok_table, ((0, 0), (0, dp - d)))
    tok_emb = _sc_gather(n_tot, dp, ch=400)(idx, tok_pad)

    w_pad = jnp.pad(W, ((0, dp - d), (0, 0)))
    pos_pad = jnp.pad(pos_table, ((0, 0), (0, dp - d)))
    return _tc_head(bx, tx, dp, vocab, g=32)(
        tok_emb, pos_pad, w_pad, b.reshape(1, vocab)
    )
